# Initial kernel scaffold; baseline (speedup 1.0000x reference)
import jax
import jax.numpy as jnp
from jax.experimental import pallas as pl


def kernel(x, table):
    raise NotImplementedError("write your pallas kernel here")


# speedup vs baseline: 2.5201x; 2.5201x over previous
import functools

import jax
import jax.numpy as jnp
from jax import lax
from jax.experimental import pallas as pl
from jax.experimental.pallas import tpu as pltpu
from jax.experimental.pallas import tpu_sc as plsc

_N = 100000
_V = 21
_D = 64
_DP = 128
_C = 200
_NCHUNKS = _N // _C
_NW = 32


def kernel(x, table):
    idx = x.reshape(_N)
    ptable = jnp.pad(table, ((0, 0), (0, _DP - _D)))
    mesh = plsc.VectorSubcoreMesh(core_axis_name="c", subcore_axis_name="s")

    @functools.partial(
        pl.kernel,
        out_type=jax.ShapeDtypeStruct((_N, _D), table.dtype),
        mesh=mesh,
        scratch_types=[
            pltpu.VMEM_SHARED((_V, _DP), jnp.float32),
            pltpu.VMEM((_C,), jnp.int32),
            pltpu.VMEM((_C, _DP), jnp.float32),
            pltpu.VMEM((_C, _D), jnp.float32),
            pltpu.SemaphoreType.DMA,
        ],
    )
    def gather_kernel(table_hbm, idx_hbm, out_hbm, tab_v, idx_v, rows_v, out64_v, sem):
        sid = lax.axis_index("s")
        wid = sid * 2 + lax.axis_index("c")

        @pl.when(sid == 0)
        def _():
            pltpu.sync_copy(table_hbm, tab_v)

        plsc.subcore_barrier()

        @pl.loop(wid, _NCHUNKS, step=_NW)
        def _(c):
            base = c * _C
            pltpu.sync_copy(idx_hbm.at[pl.ds(base, _C)], idx_v)
            pltpu.async_copy(tab_v.at[idx_v], rows_v, sem).wait()

            @pl.loop(0, _C)
            def _(r):
                @pl.loop(0, _D, step=16)
                def _(col):
                    out64_v.at[r, pl.ds(col, 16)][...] = rows_v.at[
                        r, pl.ds(col, 16)
                    ][...]

            pltpu.sync_copy(out64_v, out_hbm.at[pl.ds(base, _C)])

    return gather_kernel(ptable, idx)
